# baseline (speedup 1.0000x reference)
import jax
import jax.numpy as jnp
from jax import lax
from jax.experimental import pallas as pl
from jax.experimental.pallas import tpu as pltpu
from jax.experimental.pallas import tpu_sc as plsc

_BATCH = 16384
_NC = 2
_NS = 16
_NW = _NC * _NS
_BPW = _BATCH // _NW
_CHUNK = 128
_NCHUNK = _BPW // _CHUNK
_LANES = 16
_HI = -65536


def _mf_body(user_hbm, movie_hbm, wp_hbm, up_hbm, b_hbm, c_hbm, out_hbm,
             idx_u, idx_m, gid_u, gid_m, w_v, u_v, b_v, c_v, out_v, sem):
    wid = lax.axis_index("s") * _NC + lax.axis_index("c")
    base = wid * _BPW

    for j in range(_NCHUNK):
        pltpu.sync_copy(user_hbm.at[pl.ds(base + j * _CHUNK, _CHUNK)],
                        idx_u.at[j])
        pltpu.sync_copy(movie_hbm.at[pl.ds(base + j * _CHUNK, _CHUNK)],
                        idx_m.at[j])

    for j in range(_NCHUNK):
        for g in range(_CHUNK // _LANES):
            s = pl.ds(g * _LANES, _LANES)
            gid_u[j, s] = jnp.right_shift(idx_u[j, s], 3)
            gid_m[j, s] = jnp.right_shift(idx_m[j, s], 3)

    bias = []
    for j in range(_NCHUNK):
        bias.append(pltpu.async_copy(b_hbm.at[idx_u.at[j]], b_v.at[j], sem))
        bias.append(pltpu.async_copy(c_hbm.at[idx_m.at[j]], c_v.at[j], sem))
    for c in bias:
        c.wait()

    iota = lax.iota(jnp.int32, _LANES)

    for p in range(2):
        rows = []
        for h in range(2):
            j = p * 2 + h
            rows.append(pltpu.async_copy(wp_hbm.at[gid_u.at[j]],
                                         w_v.at[h], sem))
            rows.append(pltpu.async_copy(up_hbm.at[gid_m.at[j]],
                                         u_v.at[h], sem))
        for c in rows:
            c.wait()

        def group_body(g, _, p=p):
            e = p * 256 + g * _LANES
            ev = e + iota
            jrow = jnp.right_shift(ev, 7)
            jcol = jnp.bitwise_and(ev, _CHUNK - 1)
            hv = jnp.right_shift(jnp.bitwise_and(ev, 255), 7)
            iu = plsc.load_gather(idx_u, [jrow, jcol])
            im = plsc.load_gather(idx_m, [jrow, jcol])
            offu = jnp.bitwise_and(iu, 7) * 16
            offm = jnp.bitwise_and(im, 7) * 16
            acc = (plsc.load_gather(b_v, [jrow, jcol])
                   + plsc.load_gather(c_v, [jrow, jcol]))
            for j in range(16):
                ww = plsc.load_gather(w_v, [hv, jcol, offu + j])
                uu = plsc.load_gather(u_v, [hv, jcol, offm + j])
                wlo = plsc.bitcast(jnp.left_shift(ww, 16), jnp.float32)
                ulo = plsc.bitcast(jnp.left_shift(uu, 16), jnp.float32)
                whi = plsc.bitcast(jnp.bitwise_and(ww, _HI), jnp.float32)
                uhi = plsc.bitcast(jnp.bitwise_and(uu, _HI), jnp.float32)
                acc = acc + wlo * ulo + whi * uhi
            out_v[pl.ds(e, _LANES)] = acc
            return 0

        lax.fori_loop(0, 256 // _LANES, group_body, 0)

    pltpu.sync_copy(out_v, out_hbm.at[pl.ds(base, _BPW)])


_mf_call = pl.kernel(
    _mf_body,
    out_type=jax.ShapeDtypeStruct((_BATCH,), jnp.float32),
    mesh=plsc.VectorSubcoreMesh(core_axis_name="c", subcore_axis_name="s"),
    compiler_params=pltpu.CompilerParams(needs_layout_passes=False),
    scratch_types=[
        pltpu.VMEM((_NCHUNK, _CHUNK), jnp.int32),
        pltpu.VMEM((_NCHUNK, _CHUNK), jnp.int32),
        pltpu.VMEM((_NCHUNK, _CHUNK), jnp.int32),
        pltpu.VMEM((_NCHUNK, _CHUNK), jnp.int32),
        pltpu.VMEM((2, _CHUNK, 128), jnp.int32),
        pltpu.VMEM((2, _CHUNK, 128), jnp.int32),
        pltpu.VMEM((_NCHUNK, _CHUNK), jnp.float32),
        pltpu.VMEM((_NCHUNK, _CHUNK), jnp.float32),
        pltpu.VMEM((_BPW,), jnp.float32),
        pltpu.SemaphoreType.DMA,
    ],
)


def _pack(T, rows):
    u = lax.bitcast_convert_type(T.T, jnp.uint32)
    r = (u + jnp.uint32(0x7FFF) + ((u >> 16) & jnp.uint32(1))) >> 16
    pt = r[0::2, :] | (r[1::2, :] << 16)
    return lax.bitcast_convert_type(pt, jnp.int32).T.reshape(rows, 128)


@jax.jit
def kernel(user, movie, W, U, B, C):
    return _mf_call(user.astype(jnp.int32), movie.astype(jnp.int32),
                    _pack(W, 125000), _pack(U, 12500),
                    B.reshape(-1), C.reshape(-1))


# speedup vs baseline: 2.3031x; 2.3031x over previous
import jax
import jax.numpy as jnp
from jax import lax
from jax.experimental import pallas as pl
from jax.experimental.pallas import tpu as pltpu
from jax.experimental.pallas import tpu_sc as plsc

_BATCH = 16384
_K = 32
_NC = 2
_NS = 16
_NW = _NC * _NS
_BPW = _BATCH // _NW
_CHUNK = 128
_NCHUNK = _BPW // _CHUNK
_LANES = 16


def _mf_body(user_hbm, movie_hbm, wg_hbm, ug_hbm, b_hbm, c_hbm, out_hbm,
             idx_u, idx_m, gid_u, gid_m, w_v, u_v, b_v, c_v, out_v, sem):
    wid = lax.axis_index("s") * _NC + lax.axis_index("c")
    base = wid * _BPW

    for j in range(_NCHUNK):
        pltpu.sync_copy(user_hbm.at[pl.ds(base + j * _CHUNK, _CHUNK)],
                        idx_u.at[j])
        pltpu.sync_copy(movie_hbm.at[pl.ds(base + j * _CHUNK, _CHUNK)],
                        idx_m.at[j])

    for j in range(_NCHUNK):
        for g in range(_CHUNK // _LANES):
            s = pl.ds(g * _LANES, _LANES)
            gid_u[j, s] = jnp.right_shift(idx_u[j, s], 3)
            gid_m[j, s] = jnp.right_shift(idx_m[j, s], 3)

    bias = []
    for j in range(_NCHUNK):
        bias.append(pltpu.async_copy(b_hbm.at[idx_u.at[j]], b_v.at[j], sem))
        bias.append(pltpu.async_copy(c_hbm.at[idx_m.at[j]], c_v.at[j], sem))
    for c in bias:
        c.wait()

    iota = lax.iota(jnp.int32, _LANES)

    for p in range(_NCHUNK):
        rw = pltpu.async_copy(wg_hbm.at[gid_u.at[p]], w_v, sem)
        ru = pltpu.async_copy(ug_hbm.at[gid_m.at[p]], u_v, sem)
        rw.wait()
        ru.wait()

        def group_body(g, _, p=p):
            e = p * _CHUNK + g * _LANES
            rows = g * _LANES + iota
            pv = jnp.full((_LANES,), p, jnp.int32)
            iu = plsc.load_gather(idx_u, [pv, rows])
            im = plsc.load_gather(idx_m, [pv, rows])
            offu = jnp.bitwise_and(iu, 7) * _K
            offm = jnp.bitwise_and(im, 7) * _K
            acc = (plsc.load_gather(b_v, [pv, rows])
                   + plsc.load_gather(c_v, [pv, rows]))
            for k in range(_K):
                wv = plsc.load_gather(w_v, [rows, offu + k])
                uv = plsc.load_gather(u_v, [rows, offm + k])
                acc = acc + wv * uv
            out_v[pl.ds(e, _LANES)] = acc
            return 0

        lax.fori_loop(0, _CHUNK // _LANES, group_body, 0)

    pltpu.sync_copy(out_v, out_hbm.at[pl.ds(base, _BPW)])


_mf_call = pl.kernel(
    _mf_body,
    out_type=jax.ShapeDtypeStruct((_BATCH,), jnp.float32),
    mesh=plsc.VectorSubcoreMesh(core_axis_name="c", subcore_axis_name="s"),
    compiler_params=pltpu.CompilerParams(needs_layout_passes=False),
    scratch_types=[
        pltpu.VMEM((_NCHUNK, _CHUNK), jnp.int32),
        pltpu.VMEM((_NCHUNK, _CHUNK), jnp.int32),
        pltpu.VMEM((_NCHUNK, _CHUNK), jnp.int32),
        pltpu.VMEM((_NCHUNK, _CHUNK), jnp.int32),
        pltpu.VMEM((_CHUNK, 8 * _K), jnp.float32),
        pltpu.VMEM((_CHUNK, 8 * _K), jnp.float32),
        pltpu.VMEM((_NCHUNK, _CHUNK), jnp.float32),
        pltpu.VMEM((_NCHUNK, _CHUNK), jnp.float32),
        pltpu.VMEM((_BPW,), jnp.float32),
        pltpu.SemaphoreType.DMA,
    ],
)


@jax.jit
def kernel(user, movie, W, U, B, C):
    return _mf_call(user.astype(jnp.int32), movie.astype(jnp.int32),
                    W.reshape(125000, 256), U.reshape(12500, 256),
                    B.reshape(-1), C.reshape(-1))
